# baseline (speedup 1.0000x reference)
import functools

import jax
import jax.numpy as jnp
from jax import lax
from jax.experimental import pallas as pl
from jax.experimental.pallas import tpu as pltpu
from jax.experimental.pallas import tpu_sc as plsc

N = 10000
N_PAD = 10240
E = 160000
E_PAD = 163840
BATCH = 128
NB = E_PAD // (16 * BATCH)
NBD = E_PAD // (2 * 16 * BATCH)
ROWS_PER_TILE = N_PAD // 16


def _sc_deg(dst_idx, ones128, zeros128):
    mesh = plsc.VectorSubcoreMesh(core_axis_name="c", subcore_axis_name="s")

    @functools.partial(
        pl.kernel,
        mesh=mesh,
        out_type=jax.ShapeDtypeStruct((2, N_PAD, 128), jnp.float32),
        scratch_types=[
            pltpu.VMEM((NBD, BATCH), jnp.int32),
            pltpu.VMEM((BATCH, 128), jnp.float32),
            pltpu.VMEM_SHARED((N_PAD, 128), jnp.float32),
        ],
    )
    def run(dst_hbm, ones_hbm, zeros_hbm, out_hbm, idx_v, ones_v, acc_sh):
        c = lax.axis_index("c")
        s = lax.axis_index("s")
        pltpu.sync_copy(dst_hbm.at[c].at[s], idx_v)
        pltpu.sync_copy(ones_hbm, ones_v)
        r0 = s * ROWS_PER_TILE
        pltpu.sync_copy(zeros_hbm.at[pl.ds(r0, ROWS_PER_TILE)],
                        acc_sh.at[pl.ds(r0, ROWS_PER_TILE)])
        plsc.subcore_barrier()

        def body(j, carry):
            pltpu.sync_copy(ones_v, acc_sh.at[idx_v.at[j]], add=True)
            return carry

        lax.fori_loop(0, NBD, body, 0)
        plsc.subcore_barrier()
        pltpu.sync_copy(acc_sh.at[pl.ds(r0, ROWS_PER_TILE)],
                        out_hbm.at[c].at[pl.ds(r0, ROWS_PER_TILE)])

    return run(dst_idx, ones128, zeros128)


def _sc_agg(hw, src_idx, dst_idx, zeros128):
    mesh = plsc.VectorSubcoreMesh(core_axis_name="c", subcore_axis_name="s")

    @functools.partial(
        pl.kernel,
        mesh=mesh,
        out_type=jax.ShapeDtypeStruct((4, N_PAD, 128), jnp.float32),
        scratch_types=[
            pltpu.VMEM((NB, BATCH), jnp.int32),
            pltpu.VMEM((BATCH,), jnp.int32),
            pltpu.VMEM((BATCH,), jnp.int32),
            pltpu.VMEM((BATCH, 128), jnp.float32),
            pltpu.VMEM((BATCH, 128), jnp.float32),
            pltpu.VMEM_SHARED((N_PAD, 128), jnp.float32),
            pltpu.SemaphoreType.DMA,
            pltpu.SemaphoreType.DMA,
            pltpu.SemaphoreType.DMA,
            pltpu.SemaphoreType.DMA,
            pltpu.SemaphoreType.DMA,
            pltpu.SemaphoreType.DMA,
        ],
    )
    def run(hw_hbm, src_hbm, dst_hbm, zeros_hbm, out_hbm,
            src_v, dstb0, dstb1, buf0, buf1, acc_sh, g0, g1, d0, d1, s0, s1):
        c = lax.axis_index("c")
        s = lax.axis_index("s")
        pltpu.sync_copy(src_hbm.at[s], src_v)
        r0 = s * ROWS_PER_TILE
        for k_i in range(2):
            k = c + 2 * k_i
            pltpu.sync_copy(zeros_hbm.at[pl.ds(r0, ROWS_PER_TILE)],
                            acc_sh.at[pl.ds(r0, ROWS_PER_TILE)])
            plsc.subcore_barrier()

            pltpu.async_copy(dst_hbm.at[s].at[0], dstb0, d0)
            pltpu.async_copy(hw_hbm.at[k].at[src_v.at[0]], buf0, g0)
            pltpu.async_copy(dst_hbm.at[s].at[1], dstb1, d1)
            pltpu.async_copy(hw_hbm.at[k].at[src_v.at[1]], buf1, g1)

            def body(j2, carry):
                b0 = 2 * j2
                pltpu.make_async_copy(dst_hbm.at[s].at[b0], dstb0, d0).wait()
                pltpu.make_async_copy(hw_hbm.at[k].at[src_v.at[b0]], buf0, g0).wait()
                pltpu.async_copy(buf0, acc_sh.at[dstb0], s0, add=True)
                pltpu.make_async_copy(dst_hbm.at[s].at[b0 + 1], dstb1, d1).wait()
                pltpu.make_async_copy(hw_hbm.at[k].at[src_v.at[b0 + 1]], buf1, g1).wait()
                pltpu.async_copy(buf1, acc_sh.at[dstb1], s1, add=True)

                @pl.when(j2 < NB // 2 - 1)
                def _():
                    pltpu.make_async_copy(buf0, acc_sh.at[dstb0], s0).wait()
                    pltpu.async_copy(dst_hbm.at[s].at[b0 + 2], dstb0, d0)
                    pltpu.async_copy(hw_hbm.at[k].at[src_v.at[b0 + 2]], buf0, g0)
                    pltpu.make_async_copy(buf1, acc_sh.at[dstb1], s1).wait()
                    pltpu.async_copy(dst_hbm.at[s].at[b0 + 3], dstb1, d1)
                    pltpu.async_copy(hw_hbm.at[k].at[src_v.at[b0 + 3]], buf1, g1)
                return carry

            lax.fori_loop(0, NB // 2, body, 0)
            pltpu.make_async_copy(buf0, acc_sh.at[dstb0], s0).wait()
            pltpu.make_async_copy(buf1, acc_sh.at[dstb1], s1).wait()
            plsc.subcore_barrier()
            pltpu.sync_copy(acc_sh.at[pl.ds(r0, ROWS_PER_TILE)],
                            out_hbm.at[k].at[pl.ds(r0, ROWS_PER_TILE)])

    return run(hw, src_idx, dst_idx, zeros128)


def _tc_proj(text_item_p, img_item_p, linear1, linear2):
    def body(t_ref, im_ref, l1_ref, l2_ref, o_ref):
        o_ref[0] = jnp.dot(t_ref[...], l1_ref[...], preferred_element_type=jnp.float32)
        o_ref[1] = jnp.dot(im_ref[...], l2_ref[...], preferred_element_type=jnp.float32)

    return pl.pallas_call(
        body,
        grid=(8,),
        in_specs=[
            pl.BlockSpec((496, 128), lambda i: (i, 0)),
            pl.BlockSpec((496, 2048), lambda i: (i, 0)),
            pl.BlockSpec((128, 64), lambda i: (0, 0)),
            pl.BlockSpec((2048, 64), lambda i: (0, 0)),
        ],
        out_specs=pl.BlockSpec((2, 496, 64), lambda i: (0, i, 0)),
        out_shape=jax.ShapeDtypeStruct((2, 3968, 64), jnp.float32),
    )(text_item_p, img_item_p, linear1, linear2)


def _tc_norm(deg_part):
    def body(d_ref, o_ref):
        deg = d_ref[0, :, 0] + d_ref[1, :, 0]
        r = lax.rsqrt(jnp.maximum(deg, 1.0))
        nrm = jnp.where(deg > 0.0, r, 0.0)
        o_ref[...] = jnp.broadcast_to(nrm[:, None], (1024, 128))

    return pl.pallas_call(
        body,
        grid=(10,),
        in_specs=[pl.BlockSpec((2, 1024, 128), lambda i: (0, i, 0))],
        out_specs=pl.BlockSpec((1024, 128), lambda i: (i, 0)),
        out_shape=jax.ShapeDtypeStruct((N_PAD, 128), jnp.float32),
    )(deg_part)


def _tc_hw0(feat, W0, norm2):
    def body(f_ref, w_ref, n_ref, o_ref):
        hw = jnp.dot(f_ref[0], w_ref[...], preferred_element_type=jnp.float32)
        o_ref[0] = hw * n_ref[...]

    return pl.pallas_call(
        body,
        grid=(4, 10),
        in_specs=[
            pl.BlockSpec((1, 1024, 64), lambda k, i: (k // 2, i, 0)),
            pl.BlockSpec((64, 128), lambda k, i: (0, k % 2)),
            pl.BlockSpec((1024, 128), lambda k, i: (i, 0)),
        ],
        out_specs=pl.BlockSpec((1, 1024, 128), lambda k, i: (k, i, 0)),
        out_shape=jax.ShapeDtypeStruct((4, N_PAD, 128), jnp.float32),
    )(feat, W0, norm2)


def _tc_mid(agg0, W1, norm2):
    def body(ae_ref, ao_ref, wa_ref, wb_ref, n_ref, t0_ref, hw1_ref):
        k = pl.program_id(0)
        n = n_ref[...]
        t0a = jnp.maximum(ae_ref[0] * n, 0.0)
        t0b = jnp.maximum(ao_ref[0] * n, 0.0)
        hw1 = (jnp.dot(t0a, wa_ref[...], preferred_element_type=jnp.float32)
               + jnp.dot(t0b, wb_ref[...], preferred_element_type=jnp.float32)) * n
        hw1_ref[0] = hw1
        t0_ref[0] = jnp.where((k % 2) == 0, t0a, t0b)

    return pl.pallas_call(
        body,
        grid=(4, 20),
        in_specs=[
            pl.BlockSpec((1, 512, 128), lambda k, i: (2 * (k // 2), i, 0)),
            pl.BlockSpec((1, 512, 128), lambda k, i: (2 * (k // 2) + 1, i, 0)),
            pl.BlockSpec((128, 128), lambda k, i: (0, k % 2)),
            pl.BlockSpec((128, 128), lambda k, i: (1, k % 2)),
            pl.BlockSpec((512, 128), lambda k, i: (i, 0)),
        ],
        out_specs=[
            pl.BlockSpec((1, 512, 128), lambda k, i: (k, i, 0)),
            pl.BlockSpec((1, 512, 128), lambda k, i: (k, i, 0)),
        ],
        out_shape=[
            jax.ShapeDtypeStruct((4, N_PAD, 128), jnp.float32),
            jax.ShapeDtypeStruct((4, N_PAD, 128), jnp.float32),
        ],
    )(agg0, agg0, W1, W1, norm2)


def _tc_final_a(agg1, t0, norm2):
    def body(a_ref, t_ref, n_ref, h_ref, st_ref):
        i = pl.program_id(1)
        h = 1.12 * t_ref[0] + jnp.maximum(a_ref[0] * n_ref[...], 0.0)
        h_ref[0] = h
        st = jnp.concatenate(
            [jnp.sum(h, axis=0)[None], jnp.sum(h * h, axis=0)[None],
             jnp.zeros((6, 128), jnp.float32)], axis=0)[None]

        @pl.when(i == 0)
        def _():
            st_ref[...] = st

        @pl.when(i != 0)
        def _():
            st_ref[...] += st

    return pl.pallas_call(
        body,
        grid=(4, 20),
        in_specs=[
            pl.BlockSpec((1, 512, 128), lambda k, i: (k, i, 0)),
            pl.BlockSpec((1, 512, 128), lambda k, i: (k, i, 0)),
            pl.BlockSpec((512, 128), lambda k, i: (i, 0)),
        ],
        out_specs=[
            pl.BlockSpec((1, 512, 128), lambda k, i: (k, i, 0)),
            pl.BlockSpec((1, 8, 128), lambda k, i: (k, 0, 0)),
        ],
        out_shape=[
            jax.ShapeDtypeStruct((4, N_PAD, 128), jnp.float32),
            jax.ShapeDtypeStruct((4, 8, 128), jnp.float32),
        ],
    )(agg1, t0, norm2)


def _tc_final_b(h4, stats, gamma4, beta4):
    def body(h_ref, st_ref, g_ref, b_ref, o_ref):
        for k in range(4):
            mean = st_ref[k, 0] * (1.0 / N)
            ex2 = st_ref[k, 1] * (1.0 / N)
            var = ex2 - mean * mean
            inv = lax.rsqrt(var + 1e-5)
            g = g_ref[k, 0]
            b = b_ref[k, 0]
            o_ref[:, 128 * k:128 * (k + 1)] = (h_ref[k] - mean) * (inv * g) + b

    return pl.pallas_call(
        body,
        grid=(25,),
        in_specs=[
            pl.BlockSpec((4, 400, 128), lambda i: (0, i, 0)),
            pl.BlockSpec((4, 8, 128), lambda i: (0, 0, 0)),
            pl.BlockSpec((4, 1, 128), lambda i: (0, 0, 0)),
            pl.BlockSpec((4, 1, 128), lambda i: (0, 0, 0)),
        ],
        out_specs=pl.BlockSpec((400, 512), lambda i: (i, 0)),
        out_shape=jax.ShapeDtypeStruct((N, 512), jnp.float32),
    )(h4, stats, gamma4, beta4)


def kernel(edge_index, preference_t, preference_v, text_item, img_item,
           linear1, linear2, W0, W1, gamma, beta):
    f32 = jnp.float32
    src = edge_index[0]
    dst = edge_index[1]
    pad_idx = jnp.full((E_PAD - E,), N, jnp.int32)
    src_p = jnp.concatenate([src, pad_idx])
    dst_p = jnp.concatenate([dst, pad_idx])
    src_t = src_p.reshape(16, NB, BATCH)
    dst_t = dst_p.reshape(16, NB, BATCH)
    dst_d = dst_p.reshape(2, 16, NBD, BATCH)
    ones128 = jnp.ones((BATCH, 128), f32)
    zeros128 = jnp.zeros((N_PAD, 128), f32)

    deg_part = _sc_deg(dst_d, ones128, zeros128)
    norm2 = _tc_norm(deg_part)

    ti_p = jnp.concatenate([text_item, jnp.zeros((6, 128), f32)], axis=0)
    ii_p = jnp.concatenate([img_item, jnp.zeros((6, 2048), f32)], axis=0)
    proj = _tc_proj(ti_p, ii_p, linear1, linear2)
    zrows = jnp.zeros((N_PAD - N, 64), f32)
    feat = jnp.stack([
        jnp.concatenate([preference_t, proj[0, :3962], zrows], axis=0),
        jnp.concatenate([preference_v, proj[1, :3962], zrows], axis=0),
    ])

    hw0 = _tc_hw0(feat, W0, norm2)
    agg0 = _sc_agg(hw0, src_t, dst_t, zeros128)
    t0, hw1 = _tc_mid(agg0, W1, norm2)
    agg1 = _sc_agg(hw1, src_t, dst_t, zeros128)
    h4, stats = _tc_final_a(agg1, t0, norm2)
    return _tc_final_b(h4, stats, gamma.reshape(4, 1, 128), beta.reshape(4, 1, 128))


# speedup vs baseline: 1.0780x; 1.0780x over previous
import functools

import jax
import jax.numpy as jnp
from jax import lax
from jax.experimental import pallas as pl
from jax.experimental.pallas import tpu as pltpu
from jax.experimental.pallas import tpu_sc as plsc

N = 10000
N_PAD = 10240
E = 160000
E_PAD = 163840
BATCH = 128
NB = E_PAD // (16 * BATCH)
NBD = E_PAD // (2 * 16 * BATCH)
ROWS_PER_TILE = N_PAD // 16


def _sc_deg(dst_idx, ones128, zeros128):
    mesh = plsc.VectorSubcoreMesh(core_axis_name="c", subcore_axis_name="s")

    @functools.partial(
        pl.kernel,
        mesh=mesh,
        out_type=jax.ShapeDtypeStruct((2, N_PAD, 128), jnp.float32),
        scratch_types=[
            pltpu.VMEM((NBD, BATCH), jnp.int32),
            pltpu.VMEM((BATCH, 128), jnp.float32),
            pltpu.VMEM_SHARED((N_PAD, 128), jnp.float32),
        ],
    )
    def run(dst_hbm, ones_hbm, zeros_hbm, out_hbm, idx_v, ones_v, acc_sh):
        c = lax.axis_index("c")
        s = lax.axis_index("s")
        pltpu.sync_copy(dst_hbm.at[s].at[pl.ds(c * NBD, NBD)], idx_v)
        pltpu.sync_copy(ones_hbm, ones_v)
        r0 = s * ROWS_PER_TILE
        pltpu.sync_copy(zeros_hbm.at[pl.ds(r0, ROWS_PER_TILE)],
                        acc_sh.at[pl.ds(r0, ROWS_PER_TILE)])
        plsc.subcore_barrier()

        def body(j, carry):
            pltpu.sync_copy(ones_v, acc_sh.at[idx_v.at[j]], add=True)
            return carry

        lax.fori_loop(0, NBD, body, 0)
        plsc.subcore_barrier()
        pltpu.sync_copy(acc_sh.at[pl.ds(r0, ROWS_PER_TILE)],
                        out_hbm.at[c].at[pl.ds(r0, ROWS_PER_TILE)])

    return run(dst_idx, ones128, zeros128)


def _sc_agg(hw, src_idx, dst_idx, zeros128):
    mesh = plsc.VectorSubcoreMesh(core_axis_name="c", subcore_axis_name="s")

    @functools.partial(
        pl.kernel,
        mesh=mesh,
        out_type=jax.ShapeDtypeStruct((2, N_PAD, 128), jnp.float32),
        scratch_types=[
            pltpu.VMEM((NB, BATCH), jnp.int32),
            pltpu.VMEM((BATCH,), jnp.int32),
            pltpu.VMEM((BATCH,), jnp.int32),
            pltpu.VMEM((BATCH, 128), jnp.float32),
            pltpu.VMEM((BATCH, 128), jnp.float32),
            pltpu.VMEM_SHARED((N_PAD, 128), jnp.float32),
            pltpu.SemaphoreType.DMA,
            pltpu.SemaphoreType.DMA,
            pltpu.SemaphoreType.DMA,
            pltpu.SemaphoreType.DMA,
            pltpu.SemaphoreType.DMA,
            pltpu.SemaphoreType.DMA,
        ],
    )
    def run(hw_hbm, src_hbm, dst_hbm, zeros_hbm, out_hbm,
            src_v, dstb0, dstb1, buf0, buf1, acc_sh, g0, g1, d0, d1, s0, s1):
        c = lax.axis_index("c")
        s = lax.axis_index("s")
        pltpu.sync_copy(src_hbm.at[s], src_v)
        r0 = s * ROWS_PER_TILE
        pltpu.sync_copy(zeros_hbm.at[pl.ds(r0, ROWS_PER_TILE)],
                        acc_sh.at[pl.ds(r0, ROWS_PER_TILE)])
        plsc.subcore_barrier()

        pltpu.async_copy(dst_hbm.at[s].at[0], dstb0, d0)
        pltpu.async_copy(hw_hbm.at[c].at[src_v.at[0]], buf0, g0)
        pltpu.async_copy(dst_hbm.at[s].at[1], dstb1, d1)
        pltpu.async_copy(hw_hbm.at[c].at[src_v.at[1]], buf1, g1)

        def body(j2, carry):
            b0 = 2 * j2
            pltpu.make_async_copy(dst_hbm.at[s].at[b0], dstb0, d0).wait()
            pltpu.make_async_copy(hw_hbm.at[c].at[src_v.at[b0]], buf0, g0).wait()
            pltpu.async_copy(buf0, acc_sh.at[dstb0], s0, add=True)
            pltpu.make_async_copy(dst_hbm.at[s].at[b0 + 1], dstb1, d1).wait()
            pltpu.make_async_copy(hw_hbm.at[c].at[src_v.at[b0 + 1]], buf1, g1).wait()
            pltpu.async_copy(buf1, acc_sh.at[dstb1], s1, add=True)

            @pl.when(j2 < NB // 2 - 1)
            def _():
                pltpu.make_async_copy(buf0, acc_sh.at[dstb0], s0).wait()
                pltpu.async_copy(dst_hbm.at[s].at[b0 + 2], dstb0, d0)
                pltpu.async_copy(hw_hbm.at[c].at[src_v.at[b0 + 2]], buf0, g0)
                pltpu.make_async_copy(buf1, acc_sh.at[dstb1], s1).wait()
                pltpu.async_copy(dst_hbm.at[s].at[b0 + 3], dstb1, d1)
                pltpu.async_copy(hw_hbm.at[c].at[src_v.at[b0 + 3]], buf1, g1)
            return carry

        lax.fori_loop(0, NB // 2, body, 0)
        pltpu.make_async_copy(buf0, acc_sh.at[dstb0], s0).wait()
        pltpu.make_async_copy(buf1, acc_sh.at[dstb1], s1).wait()
        plsc.subcore_barrier()
        pltpu.sync_copy(acc_sh.at[pl.ds(r0, ROWS_PER_TILE)],
                        out_hbm.at[c].at[pl.ds(r0, ROWS_PER_TILE)])

    return run(hw, src_idx, dst_idx, zeros128)


def _tc_proj(text_item_p, img_item_p, linear1, linear2):
    def body(t_ref, im_ref, l1_ref, l2_ref, o_ref):
        o_ref[0] = jnp.dot(t_ref[...], l1_ref[...], preferred_element_type=jnp.float32)
        o_ref[1] = jnp.dot(im_ref[...], l2_ref[...], preferred_element_type=jnp.float32)

    return pl.pallas_call(
        body,
        grid=(8,),
        in_specs=[
            pl.BlockSpec((496, 128), lambda i: (i, 0)),
            pl.BlockSpec((496, 2048), lambda i: (i, 0)),
            pl.BlockSpec((128, 64), lambda i: (0, 0)),
            pl.BlockSpec((2048, 64), lambda i: (0, 0)),
        ],
        out_specs=pl.BlockSpec((2, 496, 64), lambda i: (0, i, 0)),
        out_shape=jax.ShapeDtypeStruct((2, 3968, 64), jnp.float32),
    )(text_item_p, img_item_p, linear1, linear2)


def _tc_norm(deg_part):
    def body(d_ref, o_ref):
        deg = d_ref[0, :, 0] + d_ref[1, :, 0]
        r = lax.rsqrt(jnp.maximum(deg, 1.0))
        nrm = jnp.where(deg > 0.0, r, 0.0)
        o_ref[...] = jnp.broadcast_to(nrm[:, None], (1024, 128))

    return pl.pallas_call(
        body,
        grid=(10,),
        in_specs=[pl.BlockSpec((2, 1024, 128), lambda i: (0, i, 0))],
        out_specs=pl.BlockSpec((1024, 128), lambda i: (i, 0)),
        out_shape=jax.ShapeDtypeStruct((N_PAD, 128), jnp.float32),
    )(deg_part)


def _tc_hw0(feat_h, W0, norm2):
    def body(f_ref, w_ref, n_ref, o_ref):
        hw = jnp.dot(f_ref[...], w_ref[...], preferred_element_type=jnp.float32)
        o_ref[0] = hw * n_ref[...]

    return pl.pallas_call(
        body,
        grid=(2, 10),
        in_specs=[
            pl.BlockSpec((1024, 64), lambda k, i: (i, 0)),
            pl.BlockSpec((64, 128), lambda k, i: (0, k)),
            pl.BlockSpec((1024, 128), lambda k, i: (i, 0)),
        ],
        out_specs=pl.BlockSpec((1, 1024, 128), lambda k, i: (k, i, 0)),
        out_shape=jax.ShapeDtypeStruct((2, N_PAD, 128), jnp.float32),
    )(feat_h, W0, norm2)


def _tc_mid(agg0_h, W1, norm2):
    def body(ae_ref, ao_ref, wa_ref, wb_ref, n_ref, t0_ref, hw1_ref):
        k = pl.program_id(0)
        n = n_ref[...]
        t0a = jnp.maximum(ae_ref[0] * n, 0.0)
        t0b = jnp.maximum(ao_ref[0] * n, 0.0)
        hw1 = (jnp.dot(t0a, wa_ref[...], preferred_element_type=jnp.float32)
               + jnp.dot(t0b, wb_ref[...], preferred_element_type=jnp.float32)) * n
        hw1_ref[0] = hw1
        t0_ref[0] = jnp.where(k == 0, t0a, t0b)

    return pl.pallas_call(
        body,
        grid=(2, 20),
        in_specs=[
            pl.BlockSpec((1, 512, 128), lambda k, i: (0, i, 0)),
            pl.BlockSpec((1, 512, 128), lambda k, i: (1, i, 0)),
            pl.BlockSpec((128, 128), lambda k, i: (0, k)),
            pl.BlockSpec((128, 128), lambda k, i: (1, k)),
            pl.BlockSpec((512, 128), lambda k, i: (i, 0)),
        ],
        out_specs=[
            pl.BlockSpec((1, 512, 128), lambda k, i: (k, i, 0)),
            pl.BlockSpec((1, 512, 128), lambda k, i: (k, i, 0)),
        ],
        out_shape=[
            jax.ShapeDtypeStruct((2, N_PAD, 128), jnp.float32),
            jax.ShapeDtypeStruct((2, N_PAD, 128), jnp.float32),
        ],
    )(agg0_h, agg0_h, W1, W1, norm2)


def _tc_final_a(agg1_h, t0_h, norm2):
    def body(a_ref, t_ref, n_ref, h_ref, st_ref):
        i = pl.program_id(1)
        h = 1.12 * t_ref[0] + jnp.maximum(a_ref[0] * n_ref[...], 0.0)
        h_ref[0] = h
        st = jnp.concatenate(
            [jnp.sum(h, axis=0)[None], jnp.sum(h * h, axis=0)[None],
             jnp.zeros((6, 128), jnp.float32)], axis=0)[None]

        @pl.when(i == 0)
        def _():
            st_ref[...] = st

        @pl.when(i != 0)
        def _():
            st_ref[...] += st

    return pl.pallas_call(
        body,
        grid=(2, 20),
        in_specs=[
            pl.BlockSpec((1, 512, 128), lambda k, i: (k, i, 0)),
            pl.BlockSpec((1, 512, 128), lambda k, i: (k, i, 0)),
            pl.BlockSpec((512, 128), lambda k, i: (i, 0)),
        ],
        out_specs=[
            pl.BlockSpec((1, 512, 128), lambda k, i: (k, i, 0)),
            pl.BlockSpec((1, 8, 128), lambda k, i: (k, 0, 0)),
        ],
        out_shape=[
            jax.ShapeDtypeStruct((2, N_PAD, 128), jnp.float32),
            jax.ShapeDtypeStruct((2, 8, 128), jnp.float32),
        ],
    )(agg1_h, t0_h, norm2)


def _tc_final_b(h4_t, h4_i, stats_t, stats_i, gamma4, beta4):
    def body(ht_ref, hi_ref, st_t, st_i, g_ref, b_ref, o_ref):
        for k in range(4):
            st_ref = st_t if k < 2 else st_i
            h_ref = ht_ref if k < 2 else hi_ref
            kk = k % 2
            mean = st_ref[kk, 0] * (1.0 / N)
            ex2 = st_ref[kk, 1] * (1.0 / N)
            var = ex2 - mean * mean
            inv = lax.rsqrt(var + 1e-5)
            g = g_ref[k, 0]
            b = b_ref[k, 0]
            o_ref[:, 128 * k:128 * (k + 1)] = (h_ref[kk] - mean) * (inv * g) + b

    return pl.pallas_call(
        body,
        grid=(25,),
        in_specs=[
            pl.BlockSpec((2, 400, 128), lambda i: (0, i, 0)),
            pl.BlockSpec((2, 400, 128), lambda i: (0, i, 0)),
            pl.BlockSpec((2, 8, 128), lambda i: (0, 0, 0)),
            pl.BlockSpec((2, 8, 128), lambda i: (0, 0, 0)),
            pl.BlockSpec((4, 1, 128), lambda i: (0, 0, 0)),
            pl.BlockSpec((4, 1, 128), lambda i: (0, 0, 0)),
        ],
        out_specs=pl.BlockSpec((400, 512), lambda i: (i, 0)),
        out_shape=jax.ShapeDtypeStruct((N, 512), jnp.float32),
    )(h4_t, h4_i, stats_t, stats_i, gamma4, beta4)


def kernel(edge_index, preference_t, preference_v, text_item, img_item,
           linear1, linear2, W0, W1, gamma, beta):
    f32 = jnp.float32
    src = edge_index[0]
    dst = edge_index[1]
    pad_idx = jnp.full((E_PAD - E,), N, jnp.int32)
    src_p = jnp.concatenate([src, pad_idx])
    dst_p = jnp.concatenate([dst, pad_idx])
    src_t = src_p.reshape(16, NB, BATCH)
    dst_t = dst_p.reshape(16, NB, BATCH)
    ones128 = jnp.ones((BATCH, 128), f32)
    zeros128 = jnp.zeros((N_PAD, 128), f32)

    deg_part = _sc_deg(dst_t, ones128, zeros128)
    norm2 = _tc_norm(deg_part)

    ti_p = jnp.concatenate([text_item, jnp.zeros((6, 128), f32)], axis=0)
    ii_p = jnp.concatenate([img_item, jnp.zeros((6, 2048), f32)], axis=0)
    proj = _tc_proj(ti_p, ii_p, linear1, linear2)
    zrows = jnp.zeros((N_PAD - N, 64), f32)
    feat = jnp.stack([
        jnp.concatenate([preference_t, proj[0, :3962], zrows], axis=0),
        jnp.concatenate([preference_v, proj[1, :3962], zrows], axis=0),
    ])

    hw0_t = _tc_hw0(feat[0], W0, norm2)
    hw0_i = _tc_hw0(feat[1], W0, norm2)
    agg0_t = _sc_agg(hw0_t, src_t, dst_t, zeros128)
    t0_t, hw1_t = _tc_mid(agg0_t, W1, norm2)
    agg0_i = _sc_agg(hw0_i, src_t, dst_t, zeros128)
    t0_i, hw1_i = _tc_mid(agg0_i, W1, norm2)
    agg1_t = _sc_agg(hw1_t, src_t, dst_t, zeros128)
    h4_t, stats_t = _tc_final_a(agg1_t, t0_t, norm2)
    agg1_i = _sc_agg(hw1_i, src_t, dst_t, zeros128)
    h4_i, stats_i = _tc_final_a(agg1_i, t0_i, norm2)
    return _tc_final_b(h4_t, h4_i, stats_t, stats_i,
                       gamma.reshape(4, 1, 128), beta.reshape(4, 1, 128))


# speedup vs baseline: 1.0914x; 1.0124x over previous
import functools

import jax
import jax.numpy as jnp
from jax import lax
from jax.experimental import pallas as pl
from jax.experimental.pallas import tpu as pltpu
from jax.experimental.pallas import tpu_sc as plsc

N = 10000
N_PAD = 10240
E = 160000
E_PAD = 163840
BATCH = 128
NB = E_PAD // (16 * BATCH)
NBD = E_PAD // (2 * 16 * BATCH)
ROWS_PER_TILE = N_PAD // 16


DEGW = 128


def _sc_deg(dst_idx, ones_deg, zeros_deg):
    mesh = plsc.VectorSubcoreMesh(core_axis_name="c", subcore_axis_name="s")

    @functools.partial(
        pl.kernel,
        mesh=mesh,
        out_type=jax.ShapeDtypeStruct((2, N_PAD, DEGW), jnp.float32),
        scratch_types=[
            pltpu.VMEM((NBD, BATCH), jnp.int32),
            pltpu.VMEM((BATCH, DEGW), jnp.float32),
            pltpu.VMEM_SHARED((N_PAD, DEGW), jnp.float32),
        ],
    )
    def run(dst_hbm, ones_hbm, zeros_hbm, out_hbm, idx_v, ones_v, acc_sh):
        c = lax.axis_index("c")
        s = lax.axis_index("s")
        pltpu.sync_copy(dst_hbm.at[s].at[pl.ds(c * NBD, NBD)], idx_v)
        pltpu.sync_copy(ones_hbm, ones_v)
        r0 = s * ROWS_PER_TILE
        pltpu.sync_copy(zeros_hbm.at[pl.ds(r0, ROWS_PER_TILE)],
                        acc_sh.at[pl.ds(r0, ROWS_PER_TILE)])
        plsc.subcore_barrier()

        def body(j, carry):
            pltpu.sync_copy(ones_v, acc_sh.at[idx_v.at[j]], add=True)
            return carry

        lax.fori_loop(0, NBD, body, 0)
        plsc.subcore_barrier()
        pltpu.sync_copy(acc_sh.at[pl.ds(r0, ROWS_PER_TILE)],
                        out_hbm.at[c].at[pl.ds(r0, ROWS_PER_TILE)])

    return run(dst_idx, ones_deg, zeros_deg)


def _sc_agg(hw, src_idx, dst_idx, zeros128):
    mesh = plsc.VectorSubcoreMesh(core_axis_name="c", subcore_axis_name="s")

    @functools.partial(
        pl.kernel,
        mesh=mesh,
        out_type=jax.ShapeDtypeStruct((2, N_PAD, 128), jnp.float32),
        scratch_types=[
            pltpu.VMEM((NB, BATCH), jnp.int32),
            pltpu.VMEM((BATCH,), jnp.int32),
            pltpu.VMEM((BATCH,), jnp.int32),
            pltpu.VMEM((BATCH, 128), jnp.float32),
            pltpu.VMEM((BATCH, 128), jnp.float32),
            pltpu.VMEM_SHARED((N_PAD, 128), jnp.float32),
            pltpu.SemaphoreType.DMA,
            pltpu.SemaphoreType.DMA,
            pltpu.SemaphoreType.DMA,
            pltpu.SemaphoreType.DMA,
            pltpu.SemaphoreType.DMA,
            pltpu.SemaphoreType.DMA,
        ],
    )
    def run(hw_hbm, src_hbm, dst_hbm, zeros_hbm, out_hbm,
            src_v, dstb0, dstb1, buf0, buf1, acc_sh, g0, g1, d0, d1, s0, s1):
        c = lax.axis_index("c")
        s = lax.axis_index("s")
        pltpu.sync_copy(src_hbm.at[s], src_v)
        r0 = s * ROWS_PER_TILE

        pltpu.async_copy(dst_hbm.at[s].at[0], dstb0, d0)
        pltpu.async_copy(hw_hbm.at[c].at[src_v.at[0]], buf0, g0)
        pltpu.async_copy(dst_hbm.at[s].at[1], dstb1, d1)
        pltpu.async_copy(hw_hbm.at[c].at[src_v.at[1]], buf1, g1)

        pltpu.sync_copy(zeros_hbm.at[pl.ds(r0, ROWS_PER_TILE)],
                        acc_sh.at[pl.ds(r0, ROWS_PER_TILE)])
        plsc.subcore_barrier()

        def body(j2, carry):
            b0 = 2 * j2
            pltpu.make_async_copy(dst_hbm.at[s].at[b0], dstb0, d0).wait()
            pltpu.make_async_copy(hw_hbm.at[c].at[src_v.at[b0]], buf0, g0).wait()
            pltpu.async_copy(buf0, acc_sh.at[dstb0], s0, add=True)
            pltpu.make_async_copy(dst_hbm.at[s].at[b0 + 1], dstb1, d1).wait()
            pltpu.make_async_copy(hw_hbm.at[c].at[src_v.at[b0 + 1]], buf1, g1).wait()
            pltpu.async_copy(buf1, acc_sh.at[dstb1], s1, add=True)

            @pl.when(j2 < NB // 2 - 1)
            def _():
                pltpu.make_async_copy(buf0, acc_sh.at[dstb0], s0).wait()
                pltpu.async_copy(dst_hbm.at[s].at[b0 + 2], dstb0, d0)
                pltpu.async_copy(hw_hbm.at[c].at[src_v.at[b0 + 2]], buf0, g0)
                pltpu.make_async_copy(buf1, acc_sh.at[dstb1], s1).wait()
                pltpu.async_copy(dst_hbm.at[s].at[b0 + 3], dstb1, d1)
                pltpu.async_copy(hw_hbm.at[c].at[src_v.at[b0 + 3]], buf1, g1)
            return carry

        lax.fori_loop(0, NB // 2, body, 0)
        pltpu.make_async_copy(buf0, acc_sh.at[dstb0], s0).wait()
        pltpu.make_async_copy(buf1, acc_sh.at[dstb1], s1).wait()
        plsc.subcore_barrier()
        pltpu.sync_copy(acc_sh.at[pl.ds(r0, ROWS_PER_TILE)],
                        out_hbm.at[c].at[pl.ds(r0, ROWS_PER_TILE)])

    return run(hw, src_idx, dst_idx, zeros128)


def _tc_proj(text_item_p, img_item_p, linear1, linear2):
    def body(t_ref, im_ref, l1_ref, l2_ref, o_ref):
        o_ref[0] = jnp.dot(t_ref[...], l1_ref[...], preferred_element_type=jnp.float32)
        o_ref[1] = jnp.dot(im_ref[...], l2_ref[...], preferred_element_type=jnp.float32)

    return pl.pallas_call(
        body,
        grid=(8,),
        in_specs=[
            pl.BlockSpec((496, 128), lambda i: (i, 0)),
            pl.BlockSpec((496, 2048), lambda i: (i, 0)),
            pl.BlockSpec((128, 64), lambda i: (0, 0)),
            pl.BlockSpec((2048, 64), lambda i: (0, 0)),
        ],
        out_specs=pl.BlockSpec((2, 496, 64), lambda i: (0, i, 0)),
        out_shape=jax.ShapeDtypeStruct((2, 3968, 64), jnp.float32),
    )(text_item_p, img_item_p, linear1, linear2)


def _tc_norm(deg_part):
    def body(d_ref, o_ref):
        deg = d_ref[0, :, 0] + d_ref[1, :, 0]
        r = lax.rsqrt(jnp.maximum(deg, 1.0))
        nrm = jnp.where(deg > 0.0, r, 0.0)
        o_ref[...] = jnp.broadcast_to(nrm[:, None], (1024, 128))

    return pl.pallas_call(
        body,
        grid=(10,),
        in_specs=[pl.BlockSpec((2, 1024, DEGW), lambda i: (0, i, 0))],
        out_specs=pl.BlockSpec((1024, 128), lambda i: (i, 0)),
        out_shape=jax.ShapeDtypeStruct((N_PAD, 128), jnp.float32),
    )(deg_part)


def _tc_hw0(feat_h, W0, norm2):
    def body(f_ref, w_ref, n_ref, o_ref):
        hw = jnp.dot(f_ref[...], w_ref[...], preferred_element_type=jnp.float32)
        o_ref[0] = hw * n_ref[...]

    return pl.pallas_call(
        body,
        grid=(2, 10),
        in_specs=[
            pl.BlockSpec((1024, 64), lambda k, i: (i, 0)),
            pl.BlockSpec((64, 128), lambda k, i: (0, k)),
            pl.BlockSpec((1024, 128), lambda k, i: (i, 0)),
        ],
        out_specs=pl.BlockSpec((1, 1024, 128), lambda k, i: (k, i, 0)),
        out_shape=jax.ShapeDtypeStruct((2, N_PAD, 128), jnp.float32),
    )(feat_h, W0, norm2)


def _tc_mid(agg0_h, W1, norm2):
    def body(ae_ref, ao_ref, wa_ref, wb_ref, n_ref, t0_ref, hw1_ref):
        k = pl.program_id(0)
        n = n_ref[...]
        t0a = jnp.maximum(ae_ref[0] * n, 0.0)
        t0b = jnp.maximum(ao_ref[0] * n, 0.0)
        hw1 = (jnp.dot(t0a, wa_ref[...], preferred_element_type=jnp.float32)
               + jnp.dot(t0b, wb_ref[...], preferred_element_type=jnp.float32)) * n
        hw1_ref[0] = hw1
        t0_ref[0] = jnp.where(k == 0, t0a, t0b)

    return pl.pallas_call(
        body,
        grid=(2, 20),
        in_specs=[
            pl.BlockSpec((1, 512, 128), lambda k, i: (0, i, 0)),
            pl.BlockSpec((1, 512, 128), lambda k, i: (1, i, 0)),
            pl.BlockSpec((128, 128), lambda k, i: (0, k)),
            pl.BlockSpec((128, 128), lambda k, i: (1, k)),
            pl.BlockSpec((512, 128), lambda k, i: (i, 0)),
        ],
        out_specs=[
            pl.BlockSpec((1, 512, 128), lambda k, i: (k, i, 0)),
            pl.BlockSpec((1, 512, 128), lambda k, i: (k, i, 0)),
        ],
        out_shape=[
            jax.ShapeDtypeStruct((2, N_PAD, 128), jnp.float32),
            jax.ShapeDtypeStruct((2, N_PAD, 128), jnp.float32),
        ],
    )(agg0_h, agg0_h, W1, W1, norm2)


def _tc_final_a(agg1_h, t0_h, norm2):
    def body(a_ref, t_ref, n_ref, h_ref, st_ref):
        i = pl.program_id(1)
        h = 1.12 * t_ref[0] + jnp.maximum(a_ref[0] * n_ref[...], 0.0)
        h_ref[0] = h
        st = jnp.concatenate(
            [jnp.sum(h, axis=0)[None], jnp.sum(h * h, axis=0)[None],
             jnp.zeros((6, 128), jnp.float32)], axis=0)[None]

        @pl.when(i == 0)
        def _():
            st_ref[...] = st

        @pl.when(i != 0)
        def _():
            st_ref[...] += st

    return pl.pallas_call(
        body,
        grid=(2, 20),
        in_specs=[
            pl.BlockSpec((1, 512, 128), lambda k, i: (k, i, 0)),
            pl.BlockSpec((1, 512, 128), lambda k, i: (k, i, 0)),
            pl.BlockSpec((512, 128), lambda k, i: (i, 0)),
        ],
        out_specs=[
            pl.BlockSpec((1, 512, 128), lambda k, i: (k, i, 0)),
            pl.BlockSpec((1, 8, 128), lambda k, i: (k, 0, 0)),
        ],
        out_shape=[
            jax.ShapeDtypeStruct((2, N_PAD, 128), jnp.float32),
            jax.ShapeDtypeStruct((2, 8, 128), jnp.float32),
        ],
    )(agg1_h, t0_h, norm2)


def _tc_final_b(h4_t, h4_i, stats_t, stats_i, gamma4, beta4):
    def body(ht_ref, hi_ref, st_t, st_i, g_ref, b_ref, o_ref):
        for k in range(4):
            st_ref = st_t if k < 2 else st_i
            h_ref = ht_ref if k < 2 else hi_ref
            kk = k % 2
            mean = st_ref[kk, 0] * (1.0 / N)
            ex2 = st_ref[kk, 1] * (1.0 / N)
            var = ex2 - mean * mean
            inv = lax.rsqrt(var + 1e-5)
            g = g_ref[k, 0]
            b = b_ref[k, 0]
            o_ref[:, 128 * k:128 * (k + 1)] = (h_ref[kk] - mean) * (inv * g) + b

    return pl.pallas_call(
        body,
        grid=(25,),
        in_specs=[
            pl.BlockSpec((2, 400, 128), lambda i: (0, i, 0)),
            pl.BlockSpec((2, 400, 128), lambda i: (0, i, 0)),
            pl.BlockSpec((2, 8, 128), lambda i: (0, 0, 0)),
            pl.BlockSpec((2, 8, 128), lambda i: (0, 0, 0)),
            pl.BlockSpec((4, 1, 128), lambda i: (0, 0, 0)),
            pl.BlockSpec((4, 1, 128), lambda i: (0, 0, 0)),
        ],
        out_specs=pl.BlockSpec((400, 512), lambda i: (i, 0)),
        out_shape=jax.ShapeDtypeStruct((N, 512), jnp.float32),
    )(h4_t, h4_i, stats_t, stats_i, gamma4, beta4)


def kernel(edge_index, preference_t, preference_v, text_item, img_item,
           linear1, linear2, W0, W1, gamma, beta):
    f32 = jnp.float32
    src = edge_index[0]
    dst = edge_index[1]
    pad_idx = jnp.full((E_PAD - E,), N, jnp.int32)
    src_p = jnp.concatenate([src, pad_idx])
    dst_p = jnp.concatenate([dst, pad_idx])
    src_t = src_p.reshape(16, NB, BATCH)
    dst_t = dst_p.reshape(16, NB, BATCH)
    ones_deg = jnp.ones((BATCH, DEGW), f32)
    zeros_deg = jnp.zeros((N_PAD, DEGW), f32)
    zeros128 = jnp.zeros((N_PAD, 128), f32)

    deg_part = _sc_deg(dst_t, ones_deg, zeros_deg)
    norm2 = _tc_norm(deg_part)

    ti_p = jnp.concatenate([text_item, jnp.zeros((6, 128), f32)], axis=0)
    ii_p = jnp.concatenate([img_item, jnp.zeros((6, 2048), f32)], axis=0)
    proj = _tc_proj(ti_p, ii_p, linear1, linear2)
    zrows = jnp.zeros((N_PAD - N, 64), f32)
    feat = jnp.stack([
        jnp.concatenate([preference_t, proj[0, :3962], zrows], axis=0),
        jnp.concatenate([preference_v, proj[1, :3962], zrows], axis=0),
    ])

    hw0_t = _tc_hw0(feat[0], W0, norm2)
    hw0_i = _tc_hw0(feat[1], W0, norm2)
    agg0_t = _sc_agg(hw0_t, src_t, dst_t, zeros128)
    t0_t, hw1_t = _tc_mid(agg0_t, W1, norm2)
    agg0_i = _sc_agg(hw0_i, src_t, dst_t, zeros128)
    t0_i, hw1_i = _tc_mid(agg0_i, W1, norm2)
    agg1_t = _sc_agg(hw1_t, src_t, dst_t, zeros128)
    h4_t, stats_t = _tc_final_a(agg1_t, t0_t, norm2)
    agg1_i = _sc_agg(hw1_i, src_t, dst_t, zeros128)
    h4_i, stats_i = _tc_final_a(agg1_i, t0_i, norm2)
    return _tc_final_b(h4_t, h4_i, stats_t, stats_i,
                       gamma.reshape(4, 1, 128), beta.reshape(4, 1, 128))


# speedup vs baseline: 1.6667x; 1.5272x over previous
import functools

import jax
import jax.numpy as jnp
from jax import lax
from jax.experimental import pallas as pl
from jax.experimental.pallas import tpu as pltpu
from jax.experimental.pallas import tpu_sc as plsc

N = 10000
N_PAD = 10240
E = 160000
E_PAD = 163840
BATCH = 128
NB = E_PAD // (16 * BATCH)
NBD = E_PAD // (2 * 16 * BATCH)
ROWS_PER_TILE = N_PAD // 16


DEGW = 128


def _sc_deg(dst_idx, ones_deg, zeros_deg):
    mesh = plsc.VectorSubcoreMesh(core_axis_name="c", subcore_axis_name="s")

    @functools.partial(
        pl.kernel,
        mesh=mesh,
        out_type=jax.ShapeDtypeStruct((2, N_PAD, DEGW), jnp.float32),
        scratch_types=[
            pltpu.VMEM((NBD, BATCH), jnp.int32),
            pltpu.VMEM((BATCH, DEGW), jnp.float32),
            pltpu.VMEM_SHARED((N_PAD, DEGW), jnp.float32),
        ],
    )
    def run(dst_hbm, ones_hbm, zeros_hbm, out_hbm, idx_v, ones_v, acc_sh):
        c = lax.axis_index("c")
        s = lax.axis_index("s")
        pltpu.sync_copy(dst_hbm.at[s].at[pl.ds(c * NBD, NBD)], idx_v)
        pltpu.sync_copy(ones_hbm, ones_v)
        r0 = s * ROWS_PER_TILE
        pltpu.sync_copy(zeros_hbm.at[pl.ds(r0, ROWS_PER_TILE)],
                        acc_sh.at[pl.ds(r0, ROWS_PER_TILE)])
        plsc.subcore_barrier()

        def body(j, carry):
            pltpu.sync_copy(ones_v, acc_sh.at[idx_v.at[j]], add=True)
            return carry

        lax.fori_loop(0, NBD, body, 0)
        plsc.subcore_barrier()
        pltpu.sync_copy(acc_sh.at[pl.ds(r0, ROWS_PER_TILE)],
                        out_hbm.at[c].at[pl.ds(r0, ROWS_PER_TILE)])

    return run(dst_idx, ones_deg, zeros_deg)


AB = 80
ANB = 126
ANR = ANB // 3
E_PAD_A = 16 * ANB * AB


def _sc_agg(hw, src_idx, dst_idx, zeros128):
    mesh = plsc.VectorSubcoreMesh(core_axis_name="c", subcore_axis_name="s")

    @functools.partial(
        pl.kernel,
        mesh=mesh,
        out_type=jax.ShapeDtypeStruct((2, N_PAD, 128), jnp.float32),
        scratch_types=[
            pltpu.VMEM((ANB, AB), jnp.int32),
            pltpu.VMEM((AB,), jnp.int32),
            pltpu.VMEM((AB,), jnp.int32),
            pltpu.VMEM((AB,), jnp.int32),
            pltpu.VMEM((AB, 128), jnp.float32),
            pltpu.VMEM((AB, 128), jnp.float32),
            pltpu.VMEM((AB, 128), jnp.float32),
            pltpu.VMEM_SHARED((N_PAD, 128), jnp.float32),
            pltpu.SemaphoreType.DMA,
            pltpu.SemaphoreType.DMA,
            pltpu.SemaphoreType.DMA,
            pltpu.SemaphoreType.DMA,
            pltpu.SemaphoreType.DMA,
            pltpu.SemaphoreType.DMA,
            pltpu.SemaphoreType.DMA,
            pltpu.SemaphoreType.DMA,
            pltpu.SemaphoreType.DMA,
        ],
    )
    def run(hw_hbm, src_hbm, dst_hbm, zeros_hbm, out_hbm,
            src_v, db0, db1, db2, buf0, buf1, buf2, acc_sh,
            g0, g1, g2, d0, d1, d2, s0, s1, s2):
        c = lax.axis_index("c")
        s = lax.axis_index("s")
        dbs = (db0, db1, db2)
        bufs = (buf0, buf1, buf2)
        gs = (g0, g1, g2)
        ds = (d0, d1, d2)
        ss = (s0, s1, s2)
        pltpu.sync_copy(src_hbm.at[s], src_v)
        r0 = s * ROWS_PER_TILE

        for i in range(3):
            pltpu.async_copy(dst_hbm.at[s].at[i], dbs[i], ds[i])
            pltpu.async_copy(hw_hbm.at[c].at[src_v.at[i]], bufs[i], gs[i])

        pltpu.sync_copy(zeros_hbm.at[pl.ds(r0, ROWS_PER_TILE)],
                        acc_sh.at[pl.ds(r0, ROWS_PER_TILE)])
        plsc.subcore_barrier()

        def body(r, carry):
            b = 3 * r
            for i in range(3):
                pltpu.make_async_copy(dst_hbm.at[s].at[b + i], dbs[i], ds[i]).wait()
                pltpu.make_async_copy(hw_hbm.at[c].at[src_v.at[b + i]],
                                      bufs[i], gs[i]).wait()
                pltpu.async_copy(bufs[i], acc_sh.at[dbs[i]], ss[i], add=True)

            @pl.when(r < ANR - 1)
            def _():
                for i in range(3):
                    pltpu.make_async_copy(bufs[i], acc_sh.at[dbs[i]], ss[i]).wait()
                    pltpu.async_copy(dst_hbm.at[s].at[b + 3 + i], dbs[i], ds[i])
                    pltpu.async_copy(hw_hbm.at[c].at[src_v.at[b + 3 + i]],
                                     bufs[i], gs[i])
            return carry

        lax.fori_loop(0, ANR, body, 0)
        for i in range(3):
            pltpu.make_async_copy(bufs[i], acc_sh.at[dbs[i]], ss[i]).wait()
        plsc.subcore_barrier()
        pltpu.sync_copy(acc_sh.at[pl.ds(r0, ROWS_PER_TILE)],
                        out_hbm.at[c].at[pl.ds(r0, ROWS_PER_TILE)])

    return run(hw, src_idx, dst_idx, zeros128)


def _tc_proj(text_item_p, img_item_p, linear1, linear2):
    def body(t_ref, im_ref, l1_ref, l2_ref, o_ref):
        o_ref[0] = jnp.dot(t_ref[...], l1_ref[...], preferred_element_type=jnp.float32)
        o_ref[1] = jnp.dot(im_ref[...], l2_ref[...], preferred_element_type=jnp.float32)

    return pl.pallas_call(
        body,
        grid=(8,),
        in_specs=[
            pl.BlockSpec((496, 128), lambda i: (i, 0)),
            pl.BlockSpec((496, 2048), lambda i: (i, 0)),
            pl.BlockSpec((128, 64), lambda i: (0, 0)),
            pl.BlockSpec((2048, 64), lambda i: (0, 0)),
        ],
        out_specs=pl.BlockSpec((2, 496, 64), lambda i: (0, i, 0)),
        out_shape=jax.ShapeDtypeStruct((2, 3968, 64), jnp.float32),
    )(text_item_p, img_item_p, linear1, linear2)


def _tc_norm(deg_part):
    def body(d_ref, o_ref):
        deg = d_ref[0, :, 0] + d_ref[1, :, 0]
        r = lax.rsqrt(jnp.maximum(deg, 1.0))
        nrm = jnp.where(deg > 0.0, r, 0.0)
        o_ref[...] = jnp.broadcast_to(nrm[:, None], (1024, 128))

    return pl.pallas_call(
        body,
        grid=(10,),
        in_specs=[pl.BlockSpec((2, 1024, DEGW), lambda i: (0, i, 0))],
        out_specs=pl.BlockSpec((1024, 128), lambda i: (i, 0)),
        out_shape=jax.ShapeDtypeStruct((N_PAD, 128), jnp.float32),
    )(deg_part)


def _tc_hw0(feat_h, W0, norm2):
    def body(f_ref, w_ref, n_ref, o_ref):
        hw = jnp.dot(f_ref[...], w_ref[...], preferred_element_type=jnp.float32)
        o_ref[0] = hw * n_ref[...]

    return pl.pallas_call(
        body,
        grid=(2, 10),
        in_specs=[
            pl.BlockSpec((1024, 64), lambda k, i: (i, 0)),
            pl.BlockSpec((64, 128), lambda k, i: (0, k)),
            pl.BlockSpec((1024, 128), lambda k, i: (i, 0)),
        ],
        out_specs=pl.BlockSpec((1, 1024, 128), lambda k, i: (k, i, 0)),
        out_shape=jax.ShapeDtypeStruct((2, N_PAD, 128), jnp.float32),
    )(feat_h, W0, norm2)


def _tc_mid(agg0_h, W1, norm2):
    def body(ae_ref, ao_ref, wa_ref, wb_ref, n_ref, t0_ref, hw1_ref):
        k = pl.program_id(0)
        n = n_ref[...]
        t0a = jnp.maximum(ae_ref[0] * n, 0.0)
        t0b = jnp.maximum(ao_ref[0] * n, 0.0)
        hw1 = (jnp.dot(t0a, wa_ref[...], preferred_element_type=jnp.float32)
               + jnp.dot(t0b, wb_ref[...], preferred_element_type=jnp.float32)) * n
        hw1_ref[0] = hw1
        t0_ref[0] = jnp.where(k == 0, t0a, t0b)

    return pl.pallas_call(
        body,
        grid=(2, 20),
        in_specs=[
            pl.BlockSpec((1, 512, 128), lambda k, i: (0, i, 0)),
            pl.BlockSpec((1, 512, 128), lambda k, i: (1, i, 0)),
            pl.BlockSpec((128, 128), lambda k, i: (0, k)),
            pl.BlockSpec((128, 128), lambda k, i: (1, k)),
            pl.BlockSpec((512, 128), lambda k, i: (i, 0)),
        ],
        out_specs=[
            pl.BlockSpec((1, 512, 128), lambda k, i: (k, i, 0)),
            pl.BlockSpec((1, 512, 128), lambda k, i: (k, i, 0)),
        ],
        out_shape=[
            jax.ShapeDtypeStruct((2, N_PAD, 128), jnp.float32),
            jax.ShapeDtypeStruct((2, N_PAD, 128), jnp.float32),
        ],
    )(agg0_h, agg0_h, W1, W1, norm2)


def _tc_final_a(agg1_h, t0_h, norm2):
    def body(a_ref, t_ref, n_ref, h_ref, st_ref):
        i = pl.program_id(1)
        h = 1.12 * t_ref[0] + jnp.maximum(a_ref[0] * n_ref[...], 0.0)
        h_ref[0] = h
        st = jnp.concatenate(
            [jnp.sum(h, axis=0)[None], jnp.sum(h * h, axis=0)[None],
             jnp.zeros((6, 128), jnp.float32)], axis=0)[None]

        @pl.when(i == 0)
        def _():
            st_ref[...] = st

        @pl.when(i != 0)
        def _():
            st_ref[...] += st

    return pl.pallas_call(
        body,
        grid=(2, 20),
        in_specs=[
            pl.BlockSpec((1, 512, 128), lambda k, i: (k, i, 0)),
            pl.BlockSpec((1, 512, 128), lambda k, i: (k, i, 0)),
            pl.BlockSpec((512, 128), lambda k, i: (i, 0)),
        ],
        out_specs=[
            pl.BlockSpec((1, 512, 128), lambda k, i: (k, i, 0)),
            pl.BlockSpec((1, 8, 128), lambda k, i: (k, 0, 0)),
        ],
        out_shape=[
            jax.ShapeDtypeStruct((2, N_PAD, 128), jnp.float32),
            jax.ShapeDtypeStruct((2, 8, 128), jnp.float32),
        ],
    )(agg1_h, t0_h, norm2)


def _tc_final_b(h4_t, h4_i, stats_t, stats_i, gamma4, beta4):
    def body(ht_ref, hi_ref, st_t, st_i, g_ref, b_ref, o_ref):
        for k in range(4):
            st_ref = st_t if k < 2 else st_i
            h_ref = ht_ref if k < 2 else hi_ref
            kk = k % 2
            mean = st_ref[kk, 0] * (1.0 / N)
            ex2 = st_ref[kk, 1] * (1.0 / N)
            var = ex2 - mean * mean
            inv = lax.rsqrt(var + 1e-5)
            g = g_ref[k, 0]
            b = b_ref[k, 0]
            o_ref[:, 128 * k:128 * (k + 1)] = (h_ref[kk] - mean) * (inv * g) + b

    return pl.pallas_call(
        body,
        grid=(25,),
        in_specs=[
            pl.BlockSpec((2, 400, 128), lambda i: (0, i, 0)),
            pl.BlockSpec((2, 400, 128), lambda i: (0, i, 0)),
            pl.BlockSpec((2, 8, 128), lambda i: (0, 0, 0)),
            pl.BlockSpec((2, 8, 128), lambda i: (0, 0, 0)),
            pl.BlockSpec((4, 1, 128), lambda i: (0, 0, 0)),
            pl.BlockSpec((4, 1, 128), lambda i: (0, 0, 0)),
        ],
        out_specs=pl.BlockSpec((400, 512), lambda i: (i, 0)),
        out_shape=jax.ShapeDtypeStruct((N, 512), jnp.float32),
    )(h4_t, h4_i, stats_t, stats_i, gamma4, beta4)


def kernel(edge_index, preference_t, preference_v, text_item, img_item,
           linear1, linear2, W0, W1, gamma, beta):
    f32 = jnp.float32
    src = edge_index[0]
    dst = edge_index[1]
    pad_idx = jnp.full((E_PAD - E,), N, jnp.int32)
    src_p = jnp.concatenate([src, pad_idx])
    dst_p = jnp.concatenate([dst, pad_idx])
    dst_t = dst_p.reshape(16, NB, BATCH)
    src_a = src_p[:E_PAD_A].reshape(16, ANB, AB)
    dst_a = dst_p[:E_PAD_A].reshape(16, ANB, AB)
    ones_deg = jnp.ones((BATCH, DEGW), f32)
    zeros_deg = jnp.zeros((N_PAD, DEGW), f32)
    zeros128 = jnp.zeros((N_PAD, 128), f32)

    deg_part = _sc_deg(dst_t, ones_deg, zeros_deg)
    norm2 = _tc_norm(deg_part)

    ti_p = jnp.concatenate([text_item, jnp.zeros((6, 128), f32)], axis=0)
    ii_p = jnp.concatenate([img_item, jnp.zeros((6, 2048), f32)], axis=0)
    proj = _tc_proj(ti_p, ii_p, linear1, linear2)
    zrows = jnp.zeros((N_PAD - N, 64), f32)
    feat = jnp.stack([
        jnp.concatenate([preference_t, proj[0, :3962], zrows], axis=0),
        jnp.concatenate([preference_v, proj[1, :3962], zrows], axis=0),
    ])

    hw0_t = _tc_hw0(feat[0], W0, norm2)
    hw0_i = _tc_hw0(feat[1], W0, norm2)
    agg0_t = _sc_agg(hw0_t, src_a, dst_a, zeros128)
    t0_t, hw1_t = _tc_mid(agg0_t, W1, norm2)
    agg0_i = _sc_agg(hw0_i, src_a, dst_a, zeros128)
    t0_i, hw1_i = _tc_mid(agg0_i, W1, norm2)
    agg1_t = _sc_agg(hw1_t, src_a, dst_a, zeros128)
    h4_t, stats_t = _tc_final_a(agg1_t, t0_t, norm2)
    agg1_i = _sc_agg(hw1_i, src_a, dst_a, zeros128)
    h4_i, stats_i = _tc_final_a(agg1_i, t0_i, norm2)
    return _tc_final_b(h4_t, h4_i, stats_t, stats_i,
                       gamma.reshape(4, 1, 128), beta.reshape(4, 1, 128))


# speedup vs baseline: 1.6695x; 1.0016x over previous
import functools

import jax
import jax.numpy as jnp
from jax import lax
from jax.experimental import pallas as pl
from jax.experimental.pallas import tpu as pltpu
from jax.experimental.pallas import tpu_sc as plsc

N = 10000
N_PAD = 10240
E = 160000
E_PAD = 163840
BATCH = 128
NB = E_PAD // (16 * BATCH)
NBD = E_PAD // (2 * 16 * BATCH)
ROWS_PER_TILE = N_PAD // 16


DEGW = 128


def _sc_deg(dst_idx, ones_deg, zeros_deg):
    mesh = plsc.VectorSubcoreMesh(core_axis_name="c", subcore_axis_name="s")

    @functools.partial(
        pl.kernel,
        mesh=mesh,
        out_type=jax.ShapeDtypeStruct((2, N_PAD, DEGW), jnp.float32),
        scratch_types=[
            pltpu.VMEM((NBD, BATCH), jnp.int32),
            pltpu.VMEM((BATCH, DEGW), jnp.float32),
            pltpu.VMEM_SHARED((N_PAD, DEGW), jnp.float32),
            pltpu.SemaphoreType.DMA,
            pltpu.SemaphoreType.DMA,
            pltpu.SemaphoreType.DMA,
            pltpu.SemaphoreType.DMA,
        ],
    )
    def run(dst_hbm, ones_hbm, zeros_hbm, out_hbm, idx_v, ones_v, acc_sh,
            s0, s1, s2, s3):
        c = lax.axis_index("c")
        s = lax.axis_index("s")
        ss = (s0, s1, s2, s3)
        pltpu.sync_copy(dst_hbm.at[s].at[pl.ds(c * NBD, NBD)], idx_v)
        pltpu.sync_copy(ones_hbm, ones_v)
        r0 = s * ROWS_PER_TILE
        pltpu.sync_copy(zeros_hbm.at[pl.ds(r0, ROWS_PER_TILE)],
                        acc_sh.at[pl.ds(r0, ROWS_PER_TILE)])
        plsc.subcore_barrier()

        def body(r, carry):
            for i in range(4):
                b = 4 * r + i

                @pl.when(r > 0)
                def _():
                    pltpu.make_async_copy(ones_v, acc_sh.at[idx_v.at[b - 4]],
                                          ss[i]).wait()

                pltpu.async_copy(ones_v, acc_sh.at[idx_v.at[b]], ss[i], add=True)
            return carry

        lax.fori_loop(0, NBD // 4, body, 0)
        for i in range(4):
            pltpu.make_async_copy(ones_v, acc_sh.at[idx_v.at[NBD - 4 + i]],
                                  ss[i]).wait()
        plsc.subcore_barrier()
        pltpu.sync_copy(acc_sh.at[pl.ds(r0, ROWS_PER_TILE)],
                        out_hbm.at[c].at[pl.ds(r0, ROWS_PER_TILE)])

    return run(dst_idx, ones_deg, zeros_deg)


AB = 80
ANB = 126
ANR = ANB // 3
E_PAD_A = 16 * ANB * AB


def _sc_agg(hw, src_idx, dst_idx, zeros128):
    mesh = plsc.VectorSubcoreMesh(core_axis_name="c", subcore_axis_name="s")

    @functools.partial(
        pl.kernel,
        mesh=mesh,
        out_type=jax.ShapeDtypeStruct((2, N_PAD, 128), jnp.float32),
        scratch_types=[
            pltpu.VMEM((ANB, AB), jnp.int32),
            pltpu.VMEM((AB,), jnp.int32),
            pltpu.VMEM((AB,), jnp.int32),
            pltpu.VMEM((AB,), jnp.int32),
            pltpu.VMEM((AB, 128), jnp.float32),
            pltpu.VMEM((AB, 128), jnp.float32),
            pltpu.VMEM((AB, 128), jnp.float32),
            pltpu.VMEM_SHARED((N_PAD, 128), jnp.float32),
            pltpu.SemaphoreType.DMA,
            pltpu.SemaphoreType.DMA,
            pltpu.SemaphoreType.DMA,
            pltpu.SemaphoreType.DMA,
            pltpu.SemaphoreType.DMA,
            pltpu.SemaphoreType.DMA,
            pltpu.SemaphoreType.DMA,
            pltpu.SemaphoreType.DMA,
            pltpu.SemaphoreType.DMA,
        ],
    )
    def run(hw_hbm, src_hbm, dst_hbm, zeros_hbm, out_hbm,
            src_v, db0, db1, db2, buf0, buf1, buf2, acc_sh,
            g0, g1, g2, d0, d1, d2, s0, s1, s2):
        c = lax.axis_index("c")
        s = lax.axis_index("s")
        dbs = (db0, db1, db2)
        bufs = (buf0, buf1, buf2)
        gs = (g0, g1, g2)
        ds = (d0, d1, d2)
        ss = (s0, s1, s2)
        pltpu.sync_copy(src_hbm.at[s], src_v)
        r0 = s * ROWS_PER_TILE

        for i in range(3):
            pltpu.async_copy(dst_hbm.at[s].at[i], dbs[i], ds[i])
            pltpu.async_copy(hw_hbm.at[c].at[src_v.at[i]], bufs[i], gs[i])

        pltpu.sync_copy(zeros_hbm.at[pl.ds(r0, ROWS_PER_TILE)],
                        acc_sh.at[pl.ds(r0, ROWS_PER_TILE)])
        plsc.subcore_barrier()

        def body(r, carry):
            b = 3 * r
            for i in range(3):
                pltpu.make_async_copy(dst_hbm.at[s].at[b + i], dbs[i], ds[i]).wait()
                pltpu.make_async_copy(hw_hbm.at[c].at[src_v.at[b + i]],
                                      bufs[i], gs[i]).wait()
                pltpu.async_copy(bufs[i], acc_sh.at[dbs[i]], ss[i], add=True)

            @pl.when(r < ANR - 1)
            def _():
                for i in range(3):
                    pltpu.make_async_copy(bufs[i], acc_sh.at[dbs[i]], ss[i]).wait()
                    pltpu.async_copy(dst_hbm.at[s].at[b + 3 + i], dbs[i], ds[i])
                    pltpu.async_copy(hw_hbm.at[c].at[src_v.at[b + 3 + i]],
                                     bufs[i], gs[i])
            return carry

        lax.fori_loop(0, ANR, body, 0)
        for i in range(3):
            pltpu.make_async_copy(bufs[i], acc_sh.at[dbs[i]], ss[i]).wait()
        plsc.subcore_barrier()
        pltpu.sync_copy(acc_sh.at[pl.ds(r0, ROWS_PER_TILE)],
                        out_hbm.at[c].at[pl.ds(r0, ROWS_PER_TILE)])

    return run(hw, src_idx, dst_idx, zeros128)


def _tc_proj(text_item_p, img_item_p, linear1, linear2):
    def body(t_ref, im_ref, l1_ref, l2_ref, o_ref):
        o_ref[0] = jnp.dot(t_ref[...], l1_ref[...], preferred_element_type=jnp.float32)
        o_ref[1] = jnp.dot(im_ref[...], l2_ref[...], preferred_element_type=jnp.float32)

    return pl.pallas_call(
        body,
        grid=(8,),
        in_specs=[
            pl.BlockSpec((496, 128), lambda i: (i, 0)),
            pl.BlockSpec((496, 2048), lambda i: (i, 0)),
            pl.BlockSpec((128, 64), lambda i: (0, 0)),
            pl.BlockSpec((2048, 64), lambda i: (0, 0)),
        ],
        out_specs=pl.BlockSpec((2, 496, 64), lambda i: (0, i, 0)),
        out_shape=jax.ShapeDtypeStruct((2, 3968, 64), jnp.float32),
    )(text_item_p, img_item_p, linear1, linear2)


def _tc_norm(deg_part):
    def body(d_ref, o_ref):
        deg = d_ref[0, :, 0] + d_ref[1, :, 0]
        r = lax.rsqrt(jnp.maximum(deg, 1.0))
        nrm = jnp.where(deg > 0.0, r, 0.0)
        o_ref[...] = jnp.broadcast_to(nrm[:, None], (1024, 128))

    return pl.pallas_call(
        body,
        grid=(10,),
        in_specs=[pl.BlockSpec((2, 1024, DEGW), lambda i: (0, i, 0))],
        out_specs=pl.BlockSpec((1024, 128), lambda i: (i, 0)),
        out_shape=jax.ShapeDtypeStruct((N_PAD, 128), jnp.float32),
    )(deg_part)


def _tc_hw0(feat_h, W0, norm2):
    def body(f_ref, w_ref, n_ref, o_ref):
        hw = jnp.dot(f_ref[...], w_ref[...], preferred_element_type=jnp.float32)
        o_ref[0] = hw * n_ref[...]

    return pl.pallas_call(
        body,
        grid=(2, 10),
        in_specs=[
            pl.BlockSpec((1024, 64), lambda k, i: (i, 0)),
            pl.BlockSpec((64, 128), lambda k, i: (0, k)),
            pl.BlockSpec((1024, 128), lambda k, i: (i, 0)),
        ],
        out_specs=pl.BlockSpec((1, 1024, 128), lambda k, i: (k, i, 0)),
        out_shape=jax.ShapeDtypeStruct((2, N_PAD, 128), jnp.float32),
    )(feat_h, W0, norm2)


def _tc_mid(agg0_h, W1, norm2):
    def body(ae_ref, ao_ref, wa_ref, wb_ref, n_ref, t0_ref, hw1_ref):
        k = pl.program_id(0)
        n = n_ref[...]
        t0a = jnp.maximum(ae_ref[0] * n, 0.0)
        t0b = jnp.maximum(ao_ref[0] * n, 0.0)
        hw1 = (jnp.dot(t0a, wa_ref[...], preferred_element_type=jnp.float32)
               + jnp.dot(t0b, wb_ref[...], preferred_element_type=jnp.float32)) * n
        hw1_ref[0] = hw1
        t0_ref[0] = jnp.where(k == 0, t0a, t0b)

    return pl.pallas_call(
        body,
        grid=(2, 20),
        in_specs=[
            pl.BlockSpec((1, 512, 128), lambda k, i: (0, i, 0)),
            pl.BlockSpec((1, 512, 128), lambda k, i: (1, i, 0)),
            pl.BlockSpec((128, 128), lambda k, i: (0, k)),
            pl.BlockSpec((128, 128), lambda k, i: (1, k)),
            pl.BlockSpec((512, 128), lambda k, i: (i, 0)),
        ],
        out_specs=[
            pl.BlockSpec((1, 512, 128), lambda k, i: (k, i, 0)),
            pl.BlockSpec((1, 512, 128), lambda k, i: (k, i, 0)),
        ],
        out_shape=[
            jax.ShapeDtypeStruct((2, N_PAD, 128), jnp.float32),
            jax.ShapeDtypeStruct((2, N_PAD, 128), jnp.float32),
        ],
    )(agg0_h, agg0_h, W1, W1, norm2)


def _tc_final_a(agg1_h, t0_h, norm2):
    def body(a_ref, t_ref, n_ref, h_ref, st_ref):
        i = pl.program_id(1)
        h = 1.12 * t_ref[0] + jnp.maximum(a_ref[0] * n_ref[...], 0.0)
        h_ref[0] = h
        st = jnp.concatenate(
            [jnp.sum(h, axis=0)[None], jnp.sum(h * h, axis=0)[None],
             jnp.zeros((6, 128), jnp.float32)], axis=0)[None]

        @pl.when(i == 0)
        def _():
            st_ref[...] = st

        @pl.when(i != 0)
        def _():
            st_ref[...] += st

    return pl.pallas_call(
        body,
        grid=(2, 20),
        in_specs=[
            pl.BlockSpec((1, 512, 128), lambda k, i: (k, i, 0)),
            pl.BlockSpec((1, 512, 128), lambda k, i: (k, i, 0)),
            pl.BlockSpec((512, 128), lambda k, i: (i, 0)),
        ],
        out_specs=[
            pl.BlockSpec((1, 512, 128), lambda k, i: (k, i, 0)),
            pl.BlockSpec((1, 8, 128), lambda k, i: (k, 0, 0)),
        ],
        out_shape=[
            jax.ShapeDtypeStruct((2, N_PAD, 128), jnp.float32),
            jax.ShapeDtypeStruct((2, 8, 128), jnp.float32),
        ],
    )(agg1_h, t0_h, norm2)


def _tc_final_b(h4_t, h4_i, stats_t, stats_i, gamma4, beta4):
    def body(ht_ref, hi_ref, st_t, st_i, g_ref, b_ref, o_ref):
        for k in range(4):
            st_ref = st_t if k < 2 else st_i
            h_ref = ht_ref if k < 2 else hi_ref
            kk = k % 2
            mean = st_ref[kk, 0] * (1.0 / N)
            ex2 = st_ref[kk, 1] * (1.0 / N)
            var = ex2 - mean * mean
            inv = lax.rsqrt(var + 1e-5)
            g = g_ref[k, 0]
            b = b_ref[k, 0]
            o_ref[:, 128 * k:128 * (k + 1)] = (h_ref[kk] - mean) * (inv * g) + b

    return pl.pallas_call(
        body,
        grid=(25,),
        in_specs=[
            pl.BlockSpec((2, 400, 128), lambda i: (0, i, 0)),
            pl.BlockSpec((2, 400, 128), lambda i: (0, i, 0)),
            pl.BlockSpec((2, 8, 128), lambda i: (0, 0, 0)),
            pl.BlockSpec((2, 8, 128), lambda i: (0, 0, 0)),
            pl.BlockSpec((4, 1, 128), lambda i: (0, 0, 0)),
            pl.BlockSpec((4, 1, 128), lambda i: (0, 0, 0)),
        ],
        out_specs=pl.BlockSpec((400, 512), lambda i: (i, 0)),
        out_shape=jax.ShapeDtypeStruct((N, 512), jnp.float32),
    )(h4_t, h4_i, stats_t, stats_i, gamma4, beta4)


def kernel(edge_index, preference_t, preference_v, text_item, img_item,
           linear1, linear2, W0, W1, gamma, beta):
    f32 = jnp.float32
    src = edge_index[0]
    dst = edge_index[1]
    pad_idx = jnp.full((E_PAD - E,), N, jnp.int32)
    src_p = jnp.concatenate([src, pad_idx])
    dst_p = jnp.concatenate([dst, pad_idx])
    dst_t = dst_p.reshape(16, NB, BATCH)
    src_a = src_p[:E_PAD_A].reshape(16, ANB, AB)
    dst_a = dst_p[:E_PAD_A].reshape(16, ANB, AB)
    ones_deg = jnp.ones((BATCH, DEGW), f32)
    zeros_deg = jnp.zeros((N_PAD, DEGW), f32)
    zeros128 = jnp.zeros((N_PAD, 128), f32)

    deg_part = _sc_deg(dst_t, ones_deg, zeros_deg)
    norm2 = _tc_norm(deg_part)

    ti_p = jnp.concatenate([text_item, jnp.zeros((6, 128), f32)], axis=0)
    ii_p = jnp.concatenate([img_item, jnp.zeros((6, 2048), f32)], axis=0)
    proj = _tc_proj(ti_p, ii_p, linear1, linear2)
    zrows = jnp.zeros((N_PAD - N, 64), f32)
    feat = jnp.stack([
        jnp.concatenate([preference_t, proj[0, :3962], zrows], axis=0),
        jnp.concatenate([preference_v, proj[1, :3962], zrows], axis=0),
    ])

    hw0_t = _tc_hw0(feat[0], W0, norm2)
    hw0_i = _tc_hw0(feat[1], W0, norm2)
    agg0_t = _sc_agg(hw0_t, src_a, dst_a, zeros128)
    t0_t, hw1_t = _tc_mid(agg0_t, W1, norm2)
    agg0_i = _sc_agg(hw0_i, src_a, dst_a, zeros128)
    t0_i, hw1_i = _tc_mid(agg0_i, W1, norm2)
    agg1_t = _sc_agg(hw1_t, src_a, dst_a, zeros128)
    h4_t, stats_t = _tc_final_a(agg1_t, t0_t, norm2)
    agg1_i = _sc_agg(hw1_i, src_a, dst_a, zeros128)
    h4_i, stats_i = _tc_final_a(agg1_i, t0_i, norm2)
    return _tc_final_b(h4_t, h4_i, stats_t, stats_i,
                       gamma.reshape(4, 1, 128), beta.reshape(4, 1, 128))
